# baseline (speedup 1.0000x reference)
import jax
import jax.numpy as jnp
import numpy as np
from jax.experimental import pallas as pl
from jax.sharding import Mesh, PartitionSpec as P
from jax.experimental.shard_map import shard_map

N_TOKENS = 16384
EMBED = 2048
N_EXPERTS = 64
TOP_K = 8
BLK = 2048


def _router_kernel(x_ref, w_ref, b_ref, probs_ref, idx_ref):
    w = w_ref[...]
    lt = jax.lax.dot_general(
        w, x_ref[...], (((1,), (1,)), ((), ())),
        preferred_element_type=jnp.float32,
        precision=jax.lax.Precision.DEFAULT,
    ) + b_ref[...]

    iota0 = jax.lax.broadcasted_iota(jnp.int32, lt.shape, 0)
    neg = jnp.float32(-jnp.inf)
    cur = lt
    idx_rows = []
    for _ in range(TOP_K):
        m = jnp.max(cur, axis=0, keepdims=True)
        idx = jnp.min(jnp.where(cur == m, iota0, N_EXPERTS), axis=0, keepdims=True)
        cur = jnp.where(iota0 == idx, neg, cur)
        idx_rows.append(idx)
    idx_ref[...] = jnp.concatenate(idx_rows, axis=0).T

    selected = cur == neg
    mx = jnp.max(jnp.where(selected, lt, neg), axis=0, keepdims=True)
    e = jnp.where(selected, jnp.exp(lt - mx), 0.0)
    probs_ref[...] = (e / jnp.sum(e, axis=0, keepdims=True)).T


def _router_one_core(x, W, b2):
    n = x.shape[0]
    return pl.pallas_call(
        _router_kernel,
        grid=(n // BLK,),
        in_specs=[
            pl.BlockSpec((BLK, EMBED), lambda i: (i, 0)),
            pl.BlockSpec((N_EXPERTS, EMBED), lambda i: (0, 0)),
            pl.BlockSpec((N_EXPERTS, 1), lambda i: (0, 0)),
        ],
        out_specs=[
            pl.BlockSpec((BLK, N_EXPERTS), lambda i: (i, 0)),
            pl.BlockSpec((BLK, TOP_K), lambda i: (i, 0)),
        ],
        out_shape=[
            jax.ShapeDtypeStruct((n, N_EXPERTS), jnp.float32),
            jax.ShapeDtypeStruct((n, TOP_K), jnp.int32),
        ],
    )(x, W, b2)


@jax.jit
def kernel(inputs, W, b):
    b2 = b.reshape(N_EXPERTS, 1)
    devs = jax.devices()
    n_dev = 2 if len(devs) >= 2 and N_TOKENS % (2 * BLK) == 0 else 1
    if n_dev == 1:
        return tuple(_router_one_core(inputs, W, b2))
    mesh = Mesh(np.array(devs[:n_dev]), ("d",))
    f = shard_map(
        _router_one_core,
        mesh=mesh,
        in_specs=(P("d", None), P(None, None), P(None, None)),
        out_specs=(P("d", None), P("d", None)),
        check_rep=False,
    )
    return tuple(f(inputs, W, b2))


# speedup vs baseline: 9.2280x; 9.2280x over previous
import jax
import jax.numpy as jnp
from jax.experimental import pallas as pl
from jax.experimental.pallas import tpu as pltpu

N_TOKENS = 16384
EMBED = 2048
N_EXPERTS = 64
TOP_K = 8
BLK = 2048
NBLK = N_TOKENS // BLK


def _router_kernel(x_hbm, w_ref, b_ref, probs_ref, idx_ref, xb, sems):
    i = pl.program_id(0)
    p = jax.lax.rem(i, 2)

    def mkcopy(blk, slot):
        return pltpu.make_async_copy(
            x_hbm.at[pl.ds(blk * BLK, BLK), :], xb.at[slot], sems.at[slot])

    @pl.when(i == 0)
    def _():
        mkcopy(0, 0).start()

    @pl.when(i + 1 < NBLK)
    def _():
        mkcopy(i + 1, jax.lax.rem(i + 1, 2)).start()

    mkcopy(i, p).wait()
    x = xb[p]

    w = w_ref[...]
    lt = jax.lax.dot_general(
        w, x, (((1,), (1,)), ((), ())),
        preferred_element_type=jnp.float32,
        precision=jax.lax.Precision.DEFAULT,
    ) + b_ref[...]

    iota0 = jax.lax.broadcasted_iota(jnp.int32, lt.shape, 0)
    neg = jnp.float32(-jnp.inf)
    cur = lt
    idx_rows = []
    for _ in range(TOP_K):
        m = jnp.max(cur, axis=0, keepdims=True)
        idx = jnp.min(jnp.where(cur == m, iota0, N_EXPERTS), axis=0, keepdims=True)
        cur = jnp.where(iota0 == idx, neg, cur)
        idx_rows.append(idx)
    idx_ref[...] = jnp.concatenate(idx_rows, axis=0).T

    selected = cur == neg
    mx = jnp.max(jnp.where(selected, lt, neg), axis=0, keepdims=True)
    e = jnp.where(selected, jnp.exp(lt - mx), 0.0)
    probs_ref[...] = (e / jnp.sum(e, axis=0, keepdims=True)).T


@jax.jit
def kernel(inputs, W, b):
    b2 = b.reshape(N_EXPERTS, 1)
    probs, idx = pl.pallas_call(
        _router_kernel,
        grid=(NBLK,),
        in_specs=[
            pl.BlockSpec(memory_space=pl.ANY),
            pl.BlockSpec((N_EXPERTS, EMBED), lambda i: (0, 0)),
            pl.BlockSpec((N_EXPERTS, 1), lambda i: (0, 0)),
        ],
        out_specs=[
            pl.BlockSpec((BLK, N_EXPERTS), lambda i: (i, 0)),
            pl.BlockSpec((BLK, TOP_K), lambda i: (i, 0)),
        ],
        out_shape=[
            jax.ShapeDtypeStruct((N_TOKENS, N_EXPERTS), jnp.float32),
            jax.ShapeDtypeStruct((N_TOKENS, TOP_K), jnp.int32),
        ],
        scratch_shapes=[
            pltpu.VMEM((2, BLK, EMBED), jnp.float32),
            pltpu.SemaphoreType.DMA((2,)),
        ],
    )(inputs, W, b2)
    return (probs, idx)
